# Initial kernel scaffold; baseline (speedup 1.0000x reference)
import jax
import jax.numpy as jnp
from jax.experimental import pallas as pl


def kernel(x, affinity_matrix, num_spixels, Wq, Wk, Wv, ln_gamma, ln_beta):
    raise NotImplementedError("write your pallas kernel here")


# speedup vs baseline: 31.3379x; 31.3379x over previous
import functools

import jax
import jax.numpy as jnp
from jax.experimental import pallas as pl
from jax.experimental.pallas import tpu as pltpu

C = 384
HW = 50176
K_SP = 196
TOPK = 64
NH = 8
HD = C // NH
SCALE = HD ** (-0.5)

TILE = 512
N_TILES = HW // TILE

KB = 4
TB = KB * TOPK
N_ATT = K_SP // KB


def _ln_v_body(x_ref, wv_ref, g_ref, b_ref, v_ref, xnt_ref):
    xb = x_ref[...]
    mu = jnp.mean(xb, axis=0, keepdims=True)
    var = jnp.mean((xb - mu) ** 2, axis=0, keepdims=True)
    xn = (xb - mu) * jax.lax.rsqrt(var + 1e-6)
    xn = xn * g_ref[...] + b_ref[...]
    v_ref[...] = jnp.dot(wv_ref[...], xn, preferred_element_type=jnp.float32)
    xnt_ref[...] = xn.T


def _attn_body(xg_ref, s_ref, wqt_ref, wkt_ref, wvt_ref, out_ref):
    xg = xg_ref[...]
    srow = s_ref[0]
    scol = srow.reshape(TB, 1)
    q = jnp.dot(xg, wqt_ref[...], preferred_element_type=jnp.float32)
    k = jnp.dot(xg, wkt_ref[...], preferred_element_type=jnp.float32)
    v = jnp.dot(xg, wvt_ref[...], preferred_element_type=jnp.float32)
    vw = v * scol
    t_id = jax.lax.broadcasted_iota(jnp.int32, (TB, TB), 0) // TOPK
    s_id = jax.lax.broadcasted_iota(jnp.int32, (TB, TB), 1) // TOPK
    mask = t_id == s_id
    outs = []
    for h in range(NH):
        sl = slice(h * HD, (h + 1) * HD)
        logits = jax.lax.dot_general(
            q[:, sl], k[:, sl], (((1,), (1,)), ((), ())),
            preferred_element_type=jnp.float32) * SCALE
        logits = jnp.where(mask, logits, -1e30)
        m = jnp.max(logits, axis=1, keepdims=True)
        p = jnp.exp(logits - m)
        p = p / jnp.sum(p, axis=1, keepdims=True)
        outs.append(jnp.dot(p, vw[:, sl], preferred_element_type=jnp.float32))
    out_ref[...] = jnp.concatenate(outs, axis=1) * scol


def kernel(x, affinity_matrix, num_spixels, Wq, Wk, Wv, ln_gamma, ln_beta):
    x2 = x.reshape(C, HW)
    aff = affinity_matrix.reshape(K_SP, HW)

    v_map, xnt = pl.pallas_call(
        _ln_v_body,
        grid=(N_TILES,),
        in_specs=[
            pl.BlockSpec((C, TILE), lambda i: (0, i)),
            pl.BlockSpec((C, C), lambda i: (0, 0)),
            pl.BlockSpec((C, 1), lambda i: (0, 0)),
            pl.BlockSpec((C, 1), lambda i: (0, 0)),
        ],
        out_specs=[
            pl.BlockSpec((C, TILE), lambda i: (0, i)),
            pl.BlockSpec((TILE, C), lambda i: (i, 0)),
        ],
        out_shape=[
            jax.ShapeDtypeStruct((C, HW), jnp.float32),
            jax.ShapeDtypeStruct((HW, C), jnp.float32),
        ],
    )(x2, Wv, ln_gamma.reshape(C, 1), ln_beta.reshape(C, 1))

    sims, indices = jax.lax.top_k(aff, TOPK)
    idx_flat = indices.reshape(K_SP * TOPK)
    xg = xnt[idx_flat]

    contrib = pl.pallas_call(
        _attn_body,
        grid=(N_ATT,),
        in_specs=[
            pl.BlockSpec((TB, C), lambda i: (i, 0)),
            pl.BlockSpec((1, 1, TB), lambda i: (i, 0, 0)),
            pl.BlockSpec((C, C), lambda i: (0, 0)),
            pl.BlockSpec((C, C), lambda i: (0, 0)),
            pl.BlockSpec((C, C), lambda i: (0, 0)),
        ],
        out_specs=pl.BlockSpec((TB, C), lambda i: (i, 0)),
        out_shape=jax.ShapeDtypeStruct((K_SP * TOPK, C), jnp.float32),
    )(xg, sims.reshape(N_ATT, 1, TB), Wq.T, Wk.T, Wv.T)

    res = v_map.at[:, idx_flat].add(contrib.T)
    return res.reshape(1, C, 224, 224)
